# Initial kernel scaffold; baseline (speedup 1.0000x reference)
import jax
import jax.numpy as jnp
from jax.experimental import pallas as pl


def kernel(x, edge_index, edge_weight, W, b):
    raise NotImplementedError("write your pallas kernel here")


# speedup vs baseline: 5.0500x; 5.0500x over previous
import functools

import jax
import jax.numpy as jnp
from jax import lax
from jax.experimental import pallas as pl
from jax.experimental.pallas import tpu as pltpu
from jax.experimental.pallas import tpu_sc as plsc

N_NODES = 10000
N_EDGES = 320000
D = 128
DH = D // 2

NC = 2
NS = 16
EPT = N_EDGES // NS
CHUNK = 80
NCHUNK = EPT // CHUNK
ROWS_MAIN = 624
ROWS_LAST = N_NODES - (NS - 1) * ROWS_MAIN
FBH = DH // 16


def _sc_spmm(x2, src, dst, w):
    mesh = plsc.VectorSubcoreMesh(core_axis_name="c", subcore_axis_name="s")

    @functools.partial(
        pl.kernel,
        mesh=mesh,
        compiler_params=pltpu.CompilerParams(use_tc_tiling_on_sc=False),
        out_type=jax.ShapeDtypeStruct((NC, N_NODES, DH), jnp.float32),
        scratch_types=[
            pltpu.VMEM((NCHUNK, CHUNK), jnp.int32),
            pltpu.VMEM((NCHUNK, CHUNK), jnp.int32),
            pltpu.VMEM((EPT,), jnp.float32),
            pltpu.VMEM((CHUNK,), jnp.int32),
            pltpu.VMEM((CHUNK, DH), jnp.float32),
            pltpu.VMEM_SHARED((N_NODES, DH), jnp.float32),
            pltpu.SemaphoreType.DMA,
        ],
    )
    def spmm(x_hbm, src_hbm, dst_hbm, w_hbm, out_hbm,
             src_v, dst_v, w_v, idx_v, rows_v, acc_sh, sem):
        c = lax.axis_index("c")
        s = lax.axis_index("s")
        row0 = pl.multiple_of(s * ROWS_MAIN, 8)
        coff = c * N_NODES

        zero16 = jnp.zeros((16,), jnp.float32)

        def zrow(r, carry):
            for f in range(FBH):
                rows_v[r, pl.ds(f * 16, 16)] = zero16
            return carry

        lax.fori_loop(0, CHUNK, zrow, 0)
        for k in range(ROWS_MAIN // CHUNK):
            pltpu.sync_copy(
                rows_v, acc_sh.at[pl.ds(row0 + k * CHUNK, CHUNK)])

        @pl.when(s < NS - 1)
        def _():
            pltpu.sync_copy(
                rows_v.at[pl.ds(0, ROWS_MAIN % CHUNK)],
                acc_sh.at[pl.ds(row0 + (ROWS_MAIN // CHUNK) * CHUNK,
                                ROWS_MAIN % CHUNK)])

        @pl.when(s == NS - 1)
        def _():
            for k in range(ROWS_MAIN // CHUNK, ROWS_LAST // CHUNK):
                pltpu.sync_copy(
                    rows_v, acc_sh.at[pl.ds(row0 + k * CHUNK, CHUNK)])

        plsc.subcore_barrier()

        pltpu.sync_copy(src_hbm.at[s], src_v)
        pltpu.sync_copy(dst_hbm.at[s], dst_v)
        pltpu.sync_copy(w_hbm.at[s], w_v)

        def chunk_body(ci, carry):

            for g in range(CHUNK // 16):
                iv = src_v[ci, pl.ds(g * 16, 16)]
                idx_v[pl.ds(g * 16, 16)] = iv + coff

            pltpu.async_copy(x_hbm.at[idx_v], rows_v, sem).wait()

            for g in range(CHUNK // 16):
                wvec = w_v[pl.ds(ci * CHUNK + g * 16, 16)]
                for e in range(16):
                    wval = wvec[e]
                    r = g * 16 + e
                    for f in range(FBH):
                        blk = rows_v[r, pl.ds(f * 16, 16)]
                        rows_v[r, pl.ds(f * 16, 16)] = blk * wval

            pltpu.sync_copy(rows_v, acc_sh.at[dst_v.at[ci]], add=True)
            return carry

        lax.fori_loop(0, NCHUNK, chunk_body, 0)

        plsc.subcore_barrier()

        @pl.when(s < NS - 1)
        def _():
            pltpu.sync_copy(acc_sh.at[pl.ds(row0, ROWS_MAIN)],
                            out_hbm.at[c, pl.ds(row0, ROWS_MAIN)])

        @pl.when(s == NS - 1)
        def _():
            pltpu.sync_copy(acc_sh.at[pl.ds(row0, ROWS_LAST)],
                            out_hbm.at[c, pl.ds(row0, ROWS_LAST)])

    return spmm(x2, src, dst, w)


BR = 2000


def _combine_kernel(p_ref, w0_ref, w1_ref, b_ref, o_ref):
    o_ref[...] = (
        jnp.dot(p_ref[0], w0_ref[...], preferred_element_type=jnp.float32)
        + jnp.dot(p_ref[1], w1_ref[...], preferred_element_type=jnp.float32)
        + b_ref[...])


def _tc_combine(p, W0, W1, b2):
    return pl.pallas_call(
        _combine_kernel,
        grid=(N_NODES // BR,),
        in_specs=[
            pl.BlockSpec((NC, BR, DH), lambda i: (0, i, 0)),
            pl.BlockSpec((DH, D), lambda i: (0, 0)),
            pl.BlockSpec((DH, D), lambda i: (0, 0)),
            pl.BlockSpec((1, D), lambda i: (0, 0)),
        ],
        out_specs=pl.BlockSpec((BR, D), lambda i: (i, 0)),
        out_shape=jax.ShapeDtypeStruct((N_NODES, D), jnp.float32),
    )(p, W0, W1, b2)


def kernel(x, edge_index, edge_weight, W, b):
    x2 = jnp.concatenate([x[:, :DH], x[:, DH:]], axis=0)
    src = edge_index[1].astype(jnp.int32).reshape(NS, NCHUNK, CHUNK)
    dst = edge_index[0].astype(jnp.int32).reshape(NS, NCHUNK, CHUNK)
    w = edge_weight.astype(jnp.float32).reshape(NS, EPT)
    p = _sc_spmm(x2, src, dst, w)
    return _tc_combine(p, W[:DH], W[DH:], b.reshape(1, D))


# speedup vs baseline: 8.2349x; 1.6307x over previous
import functools

import jax
import jax.numpy as jnp
from jax import lax
from jax.experimental import pallas as pl
from jax.experimental.pallas import tpu as pltpu
from jax.experimental.pallas import tpu_sc as plsc

N_NODES = 10000
N_EDGES = 320000
D = 128
DH = D // 2

NC = 2
NS = 16
EPT = N_EDGES // NS
CHUNK = 80
NCHUNK = EPT // CHUNK
ROWS_MAIN = 624
ROWS_LAST = N_NODES - (NS - 1) * ROWS_MAIN
FBH = DH // 16


def _sc_spmm(x2, src, dst, w):
    mesh = plsc.VectorSubcoreMesh(core_axis_name="c", subcore_axis_name="s")

    @functools.partial(
        pl.kernel,
        mesh=mesh,
        compiler_params=pltpu.CompilerParams(use_tc_tiling_on_sc=False),
        out_type=jax.ShapeDtypeStruct((NC, N_NODES, DH), jnp.float32),
        scratch_types=[
            pltpu.VMEM((NCHUNK, CHUNK), jnp.int32),
            pltpu.VMEM((NCHUNK, CHUNK), jnp.int32),
            pltpu.VMEM((EPT,), jnp.float32),
            pltpu.VMEM((CHUNK,), jnp.int32),
            pltpu.VMEM((CHUNK,), jnp.int32),
            pltpu.VMEM((CHUNK, DH), jnp.float32),
            pltpu.VMEM((CHUNK, DH), jnp.float32),
            pltpu.VMEM_SHARED((N_NODES, DH), jnp.float32),
            pltpu.SemaphoreType.DMA,
            pltpu.SemaphoreType.DMA,
        ],
    )
    def spmm(x_hbm, src_hbm, dst_hbm, w_hbm, out_hbm,
             src_v, dst_v, w_v, idx0, idx1, rows0, rows1, acc_sh, sem0, sem1):
        c = lax.axis_index("c")
        s = lax.axis_index("s")
        row0 = pl.multiple_of(s * ROWS_MAIN, 8)
        coff = c * N_NODES
        idxb = (idx0, idx1)
        rowsb = (rows0, rows1)
        semb = (sem0, sem1)

        zero16 = jnp.zeros((16,), jnp.float32)

        def zrow(r, carry):
            for f in range(FBH):
                rows0[r, pl.ds(f * 16, 16)] = zero16
            return carry

        lax.fori_loop(0, CHUNK, zrow, 0)
        for k in range(ROWS_MAIN // CHUNK):
            pltpu.sync_copy(
                rows0, acc_sh.at[pl.ds(row0 + k * CHUNK, CHUNK)])

        @pl.when(s < NS - 1)
        def _():
            pltpu.sync_copy(
                rows0.at[pl.ds(0, ROWS_MAIN % CHUNK)],
                acc_sh.at[pl.ds(row0 + (ROWS_MAIN // CHUNK) * CHUNK,
                                ROWS_MAIN % CHUNK)])

        @pl.when(s == NS - 1)
        def _():
            for k in range(ROWS_MAIN // CHUNK, ROWS_LAST // CHUNK):
                pltpu.sync_copy(
                    rows0, acc_sh.at[pl.ds(row0 + k * CHUNK, CHUNK)])

        plsc.subcore_barrier()

        pltpu.sync_copy(src_hbm.at[s], src_v)
        pltpu.sync_copy(dst_hbm.at[s], dst_v)
        pltpu.sync_copy(w_hbm.at[s], w_v)

        def start_gather(ci, b):
            for g in range(CHUNK // 16):
                iv = src_v[ci, pl.ds(g * 16, 16)]
                idxb[b][pl.ds(g * 16, 16)] = iv + coff
            pltpu.async_copy(x_hbm.at[idxb[b]], rowsb[b], semb[b])

        start_gather(0, 0)
        start_gather(1, 1)

        def pair_body(i, carry):
            for b in range(2):
                cur = i * 2 + b
                pltpu.make_async_copy(
                    x_hbm.at[idxb[b]], rowsb[b], semb[b]).wait()

                for g in range(CHUNK // 16):
                    wvec = w_v[pl.ds(cur * CHUNK + g * 16, 16)]
                    for e in range(16):
                        wval = wvec[e]
                        r = g * 16 + e
                        for f in range(FBH):
                            blk = rowsb[b][r, pl.ds(f * 16, 16)]
                            rowsb[b][r, pl.ds(f * 16, 16)] = blk * wval

                pltpu.sync_copy(rowsb[b], acc_sh.at[dst_v.at[cur]], add=True)

                @pl.when(cur + 2 < NCHUNK)
                def _():
                    start_gather(cur + 2, b)
            return carry

        lax.fori_loop(0, NCHUNK // 2, pair_body, 0)

        plsc.subcore_barrier()

        @pl.when(s < NS - 1)
        def _():
            pltpu.sync_copy(acc_sh.at[pl.ds(row0, ROWS_MAIN)],
                            out_hbm.at[c, pl.ds(row0, ROWS_MAIN)])

        @pl.when(s == NS - 1)
        def _():
            pltpu.sync_copy(acc_sh.at[pl.ds(row0, ROWS_LAST)],
                            out_hbm.at[c, pl.ds(row0, ROWS_LAST)])

    return spmm(x2, src, dst, w)


BR = 2000


def _combine_kernel(p_ref, w0_ref, w1_ref, b_ref, o_ref):
    o_ref[...] = (
        jnp.dot(p_ref[0], w0_ref[...], preferred_element_type=jnp.float32)
        + jnp.dot(p_ref[1], w1_ref[...], preferred_element_type=jnp.float32)
        + b_ref[...])


def _tc_combine(p, W0, W1, b2):
    return pl.pallas_call(
        _combine_kernel,
        grid=(N_NODES // BR,),
        in_specs=[
            pl.BlockSpec((NC, BR, DH), lambda i: (0, i, 0)),
            pl.BlockSpec((DH, D), lambda i: (0, 0)),
            pl.BlockSpec((DH, D), lambda i: (0, 0)),
            pl.BlockSpec((1, D), lambda i: (0, 0)),
        ],
        out_specs=pl.BlockSpec((BR, D), lambda i: (i, 0)),
        out_shape=jax.ShapeDtypeStruct((N_NODES, D), jnp.float32),
    )(p, W0, W1, b2)


def kernel(x, edge_index, edge_weight, W, b):
    x2 = jnp.concatenate([x[:, :DH], x[:, DH:]], axis=0)
    src = edge_index[1].astype(jnp.int32).reshape(NS, NCHUNK, CHUNK)
    dst = edge_index[0].astype(jnp.int32).reshape(NS, NCHUNK, CHUNK)
    w = edge_weight.astype(jnp.float32).reshape(NS, EPT)
    p = _sc_spmm(x2, src, dst, w)
    return _tc_combine(p, W[:DH], W[DH:], b.reshape(1, D))
